# baseline (speedup 1.0000x reference)
import functools

import jax
import jax.numpy as jnp
from jax import lax
from jax.experimental import pallas as pl
from jax.experimental.pallas import tpu as pltpu
from jax.experimental.pallas import tpu_sc as plsc

VOCAB_ROWS = 1000000
DIM = 64

_info = plsc.get_sparse_core_info()
NC, NS, LANES = _info.num_cores, _info.num_subcores, _info.num_lanes
NW = NC * NS

B_TOTAL = 4096 * 200
IDX_COLS = 128
IDX_ROWS_TOTAL = B_TOTAL // IDX_COLS
IDX_ROWS_PER_W = IDX_ROWS_TOTAL // NW
SLAB_IDX_ROWS = 4
SLAB_ROWS = SLAB_IDX_ROWS * IDX_COLS
N_SLABS = IDX_ROWS_PER_W // SLAB_IDX_ROWS


def _gather_body(idx_hbm, table_hbm, out_hbm, idx_v, rows_v, sem):
    wid = lax.axis_index("s") * NC + lax.axis_index("c")
    row0 = wid * IDX_ROWS_PER_W
    base = wid * IDX_ROWS_PER_W * IDX_COLS
    pltpu.sync_copy(idx_hbm.at[pl.ds(row0, IDX_ROWS_PER_W)], idx_v)

    def slab(s, _):
        for j in range(SLAB_IDX_ROWS):
            pltpu.async_copy(
                table_hbm.at[idx_v.at[s * SLAB_IDX_ROWS + j]],
                rows_v.at[pl.ds(j * IDX_COLS, IDX_COLS)],
                sem,
            ).wait()
        pltpu.sync_copy(
            rows_v, out_hbm.at[pl.ds(base + s * SLAB_ROWS, SLAB_ROWS)]
        )
        return _

    lax.fori_loop(0, N_SLABS, slab, None)


@jax.jit
def _embed_gather(x_flat2d, table):
    mesh = plsc.VectorSubcoreMesh(core_axis_name="c", subcore_axis_name="s")
    run = pl.kernel(
        _gather_body,
        mesh=mesh,
        out_type=jax.ShapeDtypeStruct((B_TOTAL, DIM), jnp.float32),
        scratch_types=[
            pltpu.VMEM((IDX_ROWS_PER_W, IDX_COLS), jnp.int32),
            pltpu.VMEM((SLAB_ROWS, DIM), jnp.float32),
            pltpu.SemaphoreType.DMA,
        ],
        compiler_params=pltpu.CompilerParams(use_tc_tiling_on_sc=False),
    )
    return run(x_flat2d, table)


def kernel(x, table):
    x_flat2d = x.reshape(IDX_ROWS_TOTAL, IDX_COLS).astype(jnp.int32)
    out = _embed_gather(x_flat2d, table)
    return out.reshape(x.shape[0], x.shape[1], DIM)


# speedup vs baseline: 1.1055x; 1.1055x over previous
import jax
import jax.numpy as jnp
from jax import lax
from jax.experimental import pallas as pl
from jax.experimental.pallas import tpu as pltpu
from jax.experimental.pallas import tpu_sc as plsc

DIM = 64

_info = plsc.get_sparse_core_info()
NC, NS = _info.num_cores, _info.num_subcores
NW = NC * NS

B_TOTAL = 4096 * 200
ROWS_PER_W = B_TOTAL // NW
SLAB = 128
N_SLABS = ROWS_PER_W // SLAB
NB = 8
LAG = 4


def _gather_body(idx_hbm, table_hbm, out_hbm, idx_v, rows, gsem, wsem):
    wid = lax.axis_index("s") * NC + lax.axis_index("c")
    base = wid * ROWS_PER_W
    pltpu.sync_copy(idx_hbm.at[pl.ds(base, ROWS_PER_W)], idx_v)

    def fire_gather(s, b):
        pltpu.async_copy(
            table_hbm.at[idx_v.at[pl.ds(s * SLAB, SLAB)]],
            rows.at[b],
            gsem.at[b],
        )

    def wait_gather(b):
        pltpu.make_async_copy(
            table_hbm.at[idx_v.at[pl.ds(0, SLAB)]], rows.at[b], gsem.at[b]
        ).wait()

    def fire_write(s, b):
        pltpu.async_copy(
            rows.at[b], out_hbm.at[pl.ds(base + s * SLAB, SLAB)], wsem.at[b]
        )

    def wait_write(b):
        pltpu.make_async_copy(
            rows.at[b], out_hbm.at[pl.ds(0, SLAB)], wsem.at[b]
        ).wait()

    for s in range(LAG):
        fire_gather(s, s % NB)
    for s in range(LAG, NB):
        fire_gather(s, s % NB)
        t = s - LAG
        wait_gather(t % NB)
        fire_write(t, t % NB)

    G = (N_SLABS - NB) // NB
    def steady(g, _):
        s0 = NB + g * NB
        for u in range(NB):
            s = s0 + u
            wait_write(u)
            fire_gather(s, u)
            t = s - LAG
            wait_gather((u + LAG) % NB)
            fire_write(t, (u + LAG) % NB)
        return _

    lax.fori_loop(0, G, steady, None)

    for t in range(N_SLABS - LAG, N_SLABS):
        wait_gather(t % NB)
        fire_write(t, t % NB)
    for b in range(NB):
        wait_write(b)


@jax.jit
def _embed_gather(x_flat, table):
    mesh = plsc.VectorSubcoreMesh(core_axis_name="c", subcore_axis_name="s")
    run = pl.kernel(
        _gather_body,
        mesh=mesh,
        out_type=jax.ShapeDtypeStruct((B_TOTAL, DIM), jnp.float32),
        scratch_types=[
            pltpu.VMEM((ROWS_PER_W,), jnp.int32),
            pltpu.VMEM((NB, SLAB, DIM), jnp.float32),
            pltpu.SemaphoreType.DMA((NB,)),
            pltpu.SemaphoreType.DMA((NB,)),
        ],
        compiler_params=pltpu.CompilerParams(use_tc_tiling_on_sc=False),
    )
    return run(x_flat, table)


def kernel(x, table):
    x_flat = x.reshape(B_TOTAL).astype(jnp.int32)
    out = _embed_gather(x_flat, table)
    return out.reshape(x.shape[0], x.shape[1], DIM)


# speedup vs baseline: 1.5719x; 1.4219x over previous
import jax
import jax.numpy as jnp
from jax import lax
from jax.experimental import pallas as pl
from jax.experimental.pallas import tpu as pltpu
from jax.experimental.pallas import tpu_sc as plsc

DIM = 64

_info = plsc.get_sparse_core_info()
NC, NS = _info.num_cores, _info.num_subcores
NW = NC * NS

B_TOTAL = 4096 * 200
ROWS_PER_W = B_TOTAL // NW
SLAB = 128
N_SLABS = ROWS_PER_W // SLAB
NB = 8
LAG = 4


def _gather_body(idx_hbm, table_hbm, out_hbm, idx_v, idx2_v, rows, gsem, wsem):
    wid = lax.axis_index("s") * NC + lax.axis_index("c")
    base = wid * ROWS_PER_W
    pltpu.sync_copy(idx_hbm.at[pl.ds(base, ROWS_PER_W)], idx_v)
    for k in range(ROWS_PER_W // 16):
        idx2_v[pl.ds(k * 16, 16)] = idx_v[pl.ds(k * 16, 16)] * 2

    def fire_gather(s, b):
        pltpu.async_copy(
            table_hbm.at[idx2_v.at[pl.ds(s * SLAB, SLAB)]],
            rows.at[b],
            gsem.at[b],
        )

    def wait_gather(b):
        pltpu.make_async_copy(
            table_hbm.at[idx2_v.at[pl.ds(0, SLAB)]], rows.at[b], gsem.at[b]
        ).wait()

    def fire_write(s, b):
        pltpu.async_copy(
            rows.at[b],
            out_hbm.at[(pl.ds(base + s * SLAB, SLAB), pl.ds(0, DIM))],
            wsem.at[b],
        )

    def wait_write(b):
        pltpu.make_async_copy(
            rows.at[b], out_hbm.at[(pl.ds(0, SLAB), pl.ds(0, DIM))], wsem.at[b]
        ).wait()

    for s in range(LAG):
        fire_gather(s, s % NB)
    for s in range(LAG, NB):
        fire_gather(s, s % NB)
        t = s - LAG
        wait_gather(t % NB)
        fire_write(t, t % NB)

    G = (N_SLABS - NB) // NB
    def steady(g, _):
        s0 = NB + g * NB
        for u in range(NB):
            s = s0 + u
            wait_write(u)
            fire_gather(s, u)
            t = s - LAG
            wait_gather((u + LAG) % NB)
            fire_write(t, (u + LAG) % NB)
        return _

    lax.fori_loop(0, G, steady, None)

    for t in range(N_SLABS - LAG, N_SLABS):
        wait_gather(t % NB)
        fire_write(t, t % NB)
    for b in range(NB):
        wait_write(b)


@jax.jit
def _embed_gather(x_flat, table_pad):
    mesh = plsc.VectorSubcoreMesh(core_axis_name="c", subcore_axis_name="s")
    run = pl.kernel(
        _gather_body,
        mesh=mesh,
        out_type=jax.ShapeDtypeStruct((B_TOTAL, 2 * DIM), jnp.float32),
        scratch_types=[
            pltpu.VMEM((ROWS_PER_W,), jnp.int32),
            pltpu.VMEM((ROWS_PER_W,), jnp.int32),
            pltpu.VMEM((NB, SLAB, DIM), jnp.float32),
            pltpu.SemaphoreType.DMA((NB,)),
            pltpu.SemaphoreType.DMA((NB,)),
        ],
        compiler_params=pltpu.CompilerParams(use_tc_tiling_on_sc=False),
    )
    return run(x_flat, table_pad)


def kernel(x, table):
    x_flat = x.reshape(B_TOTAL).astype(jnp.int32)
    table_pad = jnp.pad(table, ((0, 0), (0, DIM))).reshape(2 * 1000000, DIM)
    out = _embed_gather(x_flat, table_pad)
    return out.reshape(x.shape[0], x.shape[1], 2 * DIM)[:, :, :DIM]


# speedup vs baseline: 1.5752x; 1.0021x over previous
import jax
import jax.numpy as jnp
from jax import lax
from jax.experimental import pallas as pl
from jax.experimental.pallas import tpu as pltpu
from jax.experimental.pallas import tpu_sc as plsc

DIM = 64

_info = plsc.get_sparse_core_info()
NC, NS = _info.num_cores, _info.num_subcores
NW = NC * NS

B_TOTAL = 4096 * 200
ROWS_PER_W = B_TOTAL // NW
SLAB = 128
N_SLABS = ROWS_PER_W // SLAB
NB = 10
LAG = 5


def _gather_body(idx_hbm, table_hbm, out_hbm, idx2_v, rows, gsem, wsem):
    wid = lax.axis_index("s") * NC + lax.axis_index("c")
    base = wid * ROWS_PER_W
    pltpu.sync_copy(idx_hbm.at[pl.ds(base, ROWS_PER_W)], idx2_v)
    for k in range(ROWS_PER_W // 16):
        idx2_v[pl.ds(k * 16, 16)] = idx2_v[pl.ds(k * 16, 16)] * 2

    def fire_gather(s, b):
        pltpu.async_copy(
            table_hbm.at[idx2_v.at[pl.ds(s * SLAB, SLAB)]],
            rows.at[b],
            gsem.at[b],
        )

    def wait_gather(b):
        pltpu.make_async_copy(
            table_hbm.at[idx2_v.at[pl.ds(0, SLAB)]], rows.at[b], gsem.at[b]
        ).wait()

    def fire_write(s, b):
        pltpu.async_copy(
            rows.at[b],
            out_hbm.at[(pl.ds(base + s * SLAB, SLAB), pl.ds(0, DIM))],
            wsem.at[b],
        )

    def wait_write(b):
        pltpu.make_async_copy(
            rows.at[b], out_hbm.at[(pl.ds(0, SLAB), pl.ds(0, DIM))], wsem.at[b]
        ).wait()

    for s in range(LAG):
        fire_gather(s, s % NB)
    for s in range(LAG, NB):
        fire_gather(s, s % NB)
        t = s - LAG
        wait_gather(t % NB)
        fire_write(t, t % NB)

    G = (N_SLABS - NB) // NB
    def steady(g, _):
        s0 = NB + g * NB
        for u in range(NB):
            s = s0 + u
            wait_write(u)
            fire_gather(s, u)
            t = s - LAG
            wait_gather((u + LAG) % NB)
            fire_write(t, (u + LAG) % NB)
        return _

    lax.fori_loop(0, G, steady, None)

    for t in range(N_SLABS - LAG, N_SLABS):
        wait_gather(t % NB)
        fire_write(t, t % NB)
    for b in range(NB):
        wait_write(b)


@jax.jit
def _embed_gather(x_flat, table_pad):
    mesh = plsc.VectorSubcoreMesh(core_axis_name="c", subcore_axis_name="s")
    run = pl.kernel(
        _gather_body,
        mesh=mesh,
        out_type=jax.ShapeDtypeStruct((B_TOTAL, 2 * DIM), jnp.float32),
        scratch_types=[
            pltpu.VMEM((ROWS_PER_W,), jnp.int32),
            pltpu.VMEM((NB, SLAB, DIM), jnp.float32),
            pltpu.SemaphoreType.DMA((NB,)),
            pltpu.SemaphoreType.DMA((NB,)),
        ],
        compiler_params=pltpu.CompilerParams(use_tc_tiling_on_sc=False),
    )
    return run(x_flat, table_pad)


def kernel(x, table):
    x_flat = x.reshape(B_TOTAL).astype(jnp.int32)
    table_pad = jnp.pad(table, ((0, 0), (0, DIM))).reshape(2 * 1000000, DIM)
    out = _embed_gather(x_flat, table_pad)
    return out.reshape(x.shape[0], x.shape[1], 2 * DIM)[:, :, :DIM]
